# baseline (speedup 1.0000x reference)
import functools

import jax
import jax.numpy as jnp
from jax import lax
from jax.experimental import pallas as pl
from jax.experimental.pallas import tpu as pltpu
from jax.experimental.pallas import tpu_sc as plsc

NUM_SEG = 16
TOTAL = 32768
D = 128
DW = D + 16
NW = 32
PER_W = TOTAL // NW
RB = 256
NB = PER_W // RB
NCH = D // 16
NGRP = PER_W // 16


def _sc_partials(flat, ids):
    mesh = plsc.VectorSubcoreMesh(core_axis_name="c", subcore_axis_name="s")

    @functools.partial(
        pl.kernel,
        out_type=jax.ShapeDtypeStruct((NW, NUM_SEG, DW), jnp.float32),
        mesh=mesh,
        compiler_params=pltpu.CompilerParams(needs_layout_passes=False),
        scratch_types=[
            pltpu.VMEM((PER_W,), jnp.int32),
            pltpu.VMEM((RB, D), jnp.float32),
            pltpu.VMEM((RB, D), jnp.float32),
            pltpu.VMEM((RB, D), jnp.float32),
            pltpu.VMEM((NUM_SEG, DW), jnp.float32),
            pltpu.SemaphoreType.DMA,
            pltpu.SemaphoreType.DMA,
            pltpu.SemaphoreType.DMA,
            pltpu.SemaphoreType.DMA,
        ],
    )
    def k(flat_hbm, ids_hbm, accs_out, idv, buf0, buf1, buf2, acc,
          sem0, sem1, sem2, semi):
        cid = lax.axis_index("c")
        sid = lax.axis_index("s")
        wid = cid * 16 + sid
        base = wid * PER_W

        idcp = pltpu.async_copy(ids_hbm.at[pl.ds(base, PER_W)], idv, semi)

        bufs = (buf0, buf1, buf2)
        sems = (sem0, sem1, sem2)
        handles = [None, None, None]
        for b in range(2):
            handles[b] = pltpu.async_copy(
                flat_hbm.at[pl.ds(base + b * RB, RB)], bufs[b], sems[b])

        zeros16 = jnp.zeros((16,), jnp.float32)
        for s in range(NUM_SEG):
            for j in range(NCH):
                acc[s, pl.ds(j * 16, 16)] = zeros16
        idcp.wait()

        def searchsorted(s):
            lo = jnp.int32(0)
            hi = jnp.int32(NGRP)
            for _ in range(6):
                mid = (lo + hi) >> 1
                leader = idv[pl.ds(mid * 16, 16)][0]
                pred = leader < s
                lo = jnp.where(pred, mid + 1, lo)
                hi = jnp.where(pred, hi, mid)
            g = jnp.maximum(lo - 1, 0)
            grp = idv[pl.ds(g * 16, 16)]
            cnt = jnp.sum((grp < s).astype(jnp.int32))
            return g * 16 + cnt

        bnd = [jnp.int32(0)]
        for s in range(1, NUM_SEG):
            bnd.append(searchsorted(s))
        bnd.append(jnp.int32(PER_W))

        for s in range(NUM_SEG):
            n = (bnd[s + 1] - bnd[s]).astype(jnp.float32)
            acc[s, pl.ds(D, 16)] = jnp.full((16,), 1.0, jnp.float32) * n

        for b in range(NB):
            if b + 2 < NB:
                handles[(b + 2) % 3] = pltpu.async_copy(
                    flat_hbm.at[pl.ds(base + (b + 2) * RB, RB)],
                    bufs[(b + 2) % 3], sems[(b + 2) % 3])
            handles[b % 3].wait()
            buf = bufs[b % 3]

            for s in range(NUM_SEG):
                lo = jnp.clip(bnd[s] - b * RB, 0, RB)
                hi = jnp.clip(bnd[s + 1] - b * RB, 0, RB)
                carry0 = tuple(zeros16 for _ in range(NCH))

                @pl.when(hi > lo)
                def _(s=s, lo=lo, hi=hi, buf=buf, carry0=carry0):
                    def body(r, c):
                        return tuple(
                            c[j] + buf[r, pl.ds(j * 16, 16)]
                            for j in range(NCH))

                    sums_s = plsc.parallel_loop(
                        lo, hi, unroll=2, carry=carry0)(body)
                    for j in range(NCH):
                        acc[s, pl.ds(j * 16, 16)] += sums_s[j]

        pltpu.sync_copy(acc, accs_out.at[wid])

    return k(flat, ids)


def _combine_body(p_ref, o_ref):
    p = p_ref[...]
    s = jnp.sum(p[:, :, :D], axis=0)
    c = jnp.sum(p[:, :, D], axis=0)
    o_ref[...] = s / jnp.maximum(c, 1.0)[:, None]


def kernel(flat, segment_ids):
    ids = segment_ids.astype(jnp.int32)
    partials = _sc_partials(flat, ids)
    return pl.pallas_call(
        _combine_body,
        out_shape=jax.ShapeDtypeStruct((NUM_SEG, D), jnp.float32),
    )(partials)


# speedup vs baseline: 1.1077x; 1.1077x over previous
import functools

import jax
import jax.numpy as jnp
from jax import lax
from jax.experimental import pallas as pl
from jax.experimental.pallas import tpu as pltpu
from jax.experimental.pallas import tpu_sc as plsc

NUM_SEG = 16
TOTAL = 32768
D = 128
DW = D + 16
NW = 32
PER_W = TOTAL // NW
RB = 256
NB = PER_W // RB
NCH = D // 16
NGRP = PER_W // 16


def _sc_partials(flat, ids):
    mesh = plsc.VectorSubcoreMesh(core_axis_name="c", subcore_axis_name="s")

    @functools.partial(
        pl.kernel,
        out_type=jax.ShapeDtypeStruct((NW, NUM_SEG, DW), jnp.float32),
        mesh=mesh,
        compiler_params=pltpu.CompilerParams(needs_layout_passes=False),
        scratch_types=[
            pltpu.VMEM((PER_W,), jnp.int32),
            pltpu.VMEM((RB, D), jnp.float32),
            pltpu.VMEM((RB, D), jnp.float32),
            pltpu.VMEM((RB, D), jnp.float32),
            pltpu.VMEM((NUM_SEG, DW), jnp.float32),
            pltpu.SemaphoreType.DMA,
            pltpu.SemaphoreType.DMA,
            pltpu.SemaphoreType.DMA,
            pltpu.SemaphoreType.DMA,
        ],
    )
    def k(flat_hbm, ids_hbm, accs_out, idv, buf0, buf1, buf2, acc,
          sem0, sem1, sem2, semi):
        cid = lax.axis_index("c")
        sid = lax.axis_index("s")
        wid = cid * 16 + sid
        base = wid * PER_W

        idcp = pltpu.async_copy(ids_hbm.at[pl.ds(base, PER_W)], idv, semi)

        bufs = (buf0, buf1, buf2)
        sems = (sem0, sem1, sem2)
        handles = [None, None, None]
        for b in range(2):
            handles[b] = pltpu.async_copy(
                flat_hbm.at[pl.ds(base + b * RB, RB)], bufs[b], sems[b])

        zeros16 = jnp.zeros((16,), jnp.float32)
        for s in range(NUM_SEG):
            for j in range(NCH):
                acc[s, pl.ds(j * 16, 16)] = zeros16
        idcp.wait()

        def searchsorted(s):
            lo = jnp.int32(0)
            hi = jnp.int32(NGRP)
            for _ in range(6):
                mid = (lo + hi) >> 1
                leader = idv[pl.ds(mid * 16, 16)][0]
                pred = leader < s
                lo = jnp.where(pred, mid + 1, lo)
                hi = jnp.where(pred, hi, mid)
            g = jnp.maximum(lo - 1, 0)
            grp = idv[pl.ds(g * 16, 16)]
            cnt = jnp.sum((grp < s).astype(jnp.int32))
            return g * 16 + cnt

        bnd = [jnp.int32(0)]
        for s in range(1, NUM_SEG):
            bnd.append(searchsorted(s))
        bnd.append(jnp.int32(PER_W))

        for s in range(NUM_SEG):
            n = (bnd[s + 1] - bnd[s]).astype(jnp.float32)
            acc[s, pl.ds(D, 16)] = jnp.full((16,), 1.0, jnp.float32) * n

        for b in range(NB):
            if b + 2 < NB:
                handles[(b + 2) % 3] = pltpu.async_copy(
                    flat_hbm.at[pl.ds(base + (b + 2) * RB, RB)],
                    bufs[(b + 2) % 3], sems[(b + 2) % 3])
            handles[b % 3].wait()
            buf = bufs[b % 3]

            for s in range(NUM_SEG):
                lo = jnp.clip(bnd[s] - b * RB, 0, RB)
                hi = jnp.clip(bnd[s + 1] - b * RB, 0, RB)
                carry0 = tuple(zeros16 for _ in range(NCH))

                @pl.when(hi > lo)
                def _(s=s, lo=lo, hi=hi, buf=buf, carry0=carry0):
                    def body(r, c):
                        return tuple(
                            c[j] + buf[r, pl.ds(j * 16, 16)]
                            for j in range(NCH))

                    sums_s = plsc.parallel_loop(lo, hi, carry=carry0)(body)
                    for j in range(NCH):
                        acc[s, pl.ds(j * 16, 16)] += sums_s[j]

        pltpu.sync_copy(acc, accs_out.at[wid])

    return k(flat, ids)


def _combine_body(p_ref, o_ref):
    p = p_ref[...]
    s = jnp.sum(p[:, :, :D], axis=0)
    c = jnp.sum(p[:, :, D], axis=0)
    o_ref[...] = s / jnp.maximum(c, 1.0)[:, None]


def kernel(flat, segment_ids):
    ids = segment_ids.astype(jnp.int32)
    partials = _sc_partials(flat, ids)
    return pl.pallas_call(
        _combine_body,
        out_shape=jax.ShapeDtypeStruct((NUM_SEG, D), jnp.float32),
    )(partials)


# speedup vs baseline: 1.1854x; 1.0702x over previous
import functools
import math

import jax
import jax.numpy as jnp
from jax import lax
from jax.experimental import pallas as pl
from jax.experimental.pallas import tpu as pltpu
from jax.experimental.pallas import tpu_sc as plsc

NUM_SEG = 16
TOTAL = 32768
D = 128
DW = D + 16
NW = 32

SPLIT = 16384
PER_W = SPLIT // NW
RB = 256
NB = PER_W // RB
NCH = D // 16
NGRP = PER_W // 16
NSTEP = int(math.log2(NGRP))

TCB = 2048
NTC = (TOTAL - SPLIT) // TCB


def _sc_partials(flat, ids):
    mesh = plsc.VectorSubcoreMesh(core_axis_name="c", subcore_axis_name="s")

    @functools.partial(
        pl.kernel,
        out_type=jax.ShapeDtypeStruct((NW, NUM_SEG, DW), jnp.float32),
        mesh=mesh,
        compiler_params=pltpu.CompilerParams(needs_layout_passes=False),
        scratch_types=[
            pltpu.VMEM((PER_W,), jnp.int32),
            pltpu.VMEM((RB, D), jnp.float32),
            pltpu.VMEM((RB, D), jnp.float32),
            pltpu.VMEM((NUM_SEG, DW), jnp.float32),
            pltpu.SemaphoreType.DMA,
            pltpu.SemaphoreType.DMA,
            pltpu.SemaphoreType.DMA,
        ],
    )
    def k(flat_hbm, ids_hbm, accs_out, idv, buf0, buf1, acc,
          sem0, sem1, semi):
        cid = lax.axis_index("c")
        sid = lax.axis_index("s")
        wid = cid * 16 + sid
        base = wid * PER_W

        idcp = pltpu.async_copy(ids_hbm.at[pl.ds(base, PER_W)], idv, semi)

        bufs = (buf0, buf1)
        sems = (sem0, sem1)
        handles = [None, None]
        handles[0] = pltpu.async_copy(
            flat_hbm.at[pl.ds(base, RB)], buf0, sem0)

        zeros16 = jnp.zeros((16,), jnp.float32)
        for s in range(NUM_SEG):
            for j in range(NCH):
                acc[s, pl.ds(j * 16, 16)] = zeros16
        idcp.wait()

        def searchsorted(s):
            lo = jnp.int32(0)
            hi = jnp.int32(NGRP)
            for _ in range(NSTEP):
                mid = (lo + hi) >> 1
                leader = idv[pl.ds(mid * 16, 16)][0]
                pred = leader < s
                lo = jnp.where(pred, mid + 1, lo)
                hi = jnp.where(pred, hi, mid)
            g = jnp.maximum(lo - 1, 0)
            grp = idv[pl.ds(g * 16, 16)]
            cnt = jnp.sum((grp < s).astype(jnp.int32))
            return g * 16 + cnt

        bnd = [jnp.int32(0)]
        for s in range(1, NUM_SEG):
            bnd.append(searchsorted(s))
        bnd.append(jnp.int32(PER_W))

        for s in range(NUM_SEG):
            n = (bnd[s + 1] - bnd[s]).astype(jnp.float32)
            acc[s, pl.ds(D, 16)] = jnp.full((16,), 1.0, jnp.float32) * n

        for b in range(NB):
            if b + 1 < NB:
                handles[(b + 1) % 2] = pltpu.async_copy(
                    flat_hbm.at[pl.ds(base + (b + 1) * RB, RB)],
                    bufs[(b + 1) % 2], sems[(b + 1) % 2])
            handles[b % 2].wait()
            buf = bufs[b % 2]

            for s in range(NUM_SEG):
                lo = jnp.clip(bnd[s] - b * RB, 0, RB)
                hi = jnp.clip(bnd[s + 1] - b * RB, 0, RB)
                carry0 = tuple(zeros16 for _ in range(NCH))

                @pl.when(hi > lo)
                def _(s=s, lo=lo, hi=hi, buf=buf, carry0=carry0):
                    def body(r, c):
                        return tuple(
                            c[j] + buf[r, pl.ds(j * 16, 16)]
                            for j in range(NCH))

                    sums_s = plsc.parallel_loop(lo, hi, carry=carry0)(body)
                    for j in range(NCH):
                        acc[s, pl.ds(j * 16, 16)] += sums_s[j]

        pltpu.sync_copy(acc, accs_out.at[wid])

    return k(flat, ids)


def _tc_body(part_ref, flat_ref, ids_ref, o_ref, acc_s, acc_c):
    g = pl.program_id(0)

    @pl.when(g == 0)
    def _():
        acc_s[...] = jnp.zeros((NUM_SEG, D), jnp.float32)
        acc_c[...] = jnp.zeros((NUM_SEG, D), jnp.float32)

    ids = ids_ref[0, 0]
    oh = (lax.broadcasted_iota(jnp.int32, (NUM_SEG, TCB), 0)
          == ids[None, :]).astype(jnp.float32)
    acc_s[...] += jnp.dot(oh, flat_ref[...],
                          preferred_element_type=jnp.float32)
    acc_c[...] += jnp.broadcast_to(
        jnp.sum(oh, axis=1, keepdims=True), (NUM_SEG, D))

    @pl.when(g == NTC - 1)
    def _():
        p = part_ref[...]
        s = acc_s[...] + jnp.sum(p[:, :, :D], axis=0)
        c = acc_c[...][:, :1] + jnp.sum(p[:, :, D], axis=0)[:, None]
        o_ref[...] = s / jnp.maximum(c, 1.0)


def kernel(flat, segment_ids):
    ids = segment_ids.astype(jnp.int32)
    partials = _sc_partials(flat, ids)
    ids3 = ids.reshape(TOTAL // TCB, 1, TCB)
    return pl.pallas_call(
        _tc_body,
        grid=(NTC,),
        in_specs=[
            pl.BlockSpec((NW, NUM_SEG, DW), lambda g: (0, 0, 0)),
            pl.BlockSpec((TCB, D), lambda g: (SPLIT // TCB + g, 0)),
            pl.BlockSpec((1, 1, TCB), lambda g: (SPLIT // TCB + g, 0, 0)),
        ],
        out_specs=pl.BlockSpec((NUM_SEG, D), lambda g: (0, 0)),
        scratch_shapes=[
            pltpu.VMEM((NUM_SEG, D), jnp.float32),
            pltpu.VMEM((NUM_SEG, D), jnp.float32),
        ],
        out_shape=jax.ShapeDtypeStruct((NUM_SEG, D), jnp.float32),
    )(partials, flat, ids3)


# speedup vs baseline: 1.3850x; 1.1684x over previous
import functools
import math

import jax
import jax.numpy as jnp
from jax import lax
from jax.experimental import pallas as pl
from jax.experimental.pallas import tpu as pltpu
from jax.experimental.pallas import tpu_sc as plsc

NUM_SEG = 16
TOTAL = 32768
D = 128
DW = D + 16
NW = 32

SPLIT = 16384
PER_W = SPLIT // NW
RB = 256
NB = PER_W // RB
NCH = D // 16
NGRP = PER_W // 16
NSTEP = int(math.log2(NGRP))

TCB = 2048
NTC = (TOTAL - SPLIT) // TCB


def _sc_partials(flat, ids):
    mesh = plsc.VectorSubcoreMesh(core_axis_name="c", subcore_axis_name="s")

    @functools.partial(
        pl.kernel,
        out_type=jax.ShapeDtypeStruct((NW, NUM_SEG, DW), jnp.float32),
        mesh=mesh,
        compiler_params=pltpu.CompilerParams(needs_layout_passes=False),
        scratch_types=[
            pltpu.VMEM((PER_W,), jnp.int32),
            pltpu.VMEM((RB, D), jnp.float32),
            pltpu.VMEM((RB, D), jnp.float32),
            pltpu.VMEM((NUM_SEG, DW), jnp.float32),
            pltpu.SemaphoreType.DMA,
            pltpu.SemaphoreType.DMA,
            pltpu.SemaphoreType.DMA,
        ],
    )
    def k(flat_hbm, ids_hbm, accs_out, idv, buf0, buf1, acc,
          sem0, sem1, semi):
        cid = lax.axis_index("c")
        sid = lax.axis_index("s")
        wid = cid * 16 + sid
        base = wid * PER_W

        idcp = pltpu.async_copy(ids_hbm.at[pl.ds(base, PER_W)], idv, semi)

        bufs = (buf0, buf1)
        sems = (sem0, sem1)
        handles = [None, None]
        handles[0] = pltpu.async_copy(
            flat_hbm.at[pl.ds(base, RB)], buf0, sem0)

        zeros16 = jnp.zeros((16,), jnp.float32)
        for s in range(NUM_SEG):
            for j in range(NCH):
                acc[s, pl.ds(j * 16, 16)] = zeros16
        idcp.wait()

        def searchsorted(s):
            lo = jnp.int32(0)
            hi = jnp.int32(NGRP)
            for _ in range(NSTEP):
                mid = (lo + hi) >> 1
                leader = idv[pl.ds(mid * 16, 16)][0]
                pred = leader < s
                lo = jnp.where(pred, mid + 1, lo)
                hi = jnp.where(pred, hi, mid)
            g = jnp.maximum(lo - 1, 0)
            grp = idv[pl.ds(g * 16, 16)]
            cnt = jnp.sum((grp < s).astype(jnp.int32))
            return g * 16 + cnt

        bnd = [jnp.int32(0)]
        for s in range(1, NUM_SEG):
            bnd.append(searchsorted(s))
        bnd.append(jnp.int32(PER_W))

        for s in range(NUM_SEG):
            n = (bnd[s + 1] - bnd[s]).astype(jnp.float32)
            acc[s, pl.ds(D, 16)] = jnp.full((16,), 1.0, jnp.float32) * n

        for b in range(NB):
            if b + 1 < NB:
                handles[(b + 1) % 2] = pltpu.async_copy(
                    flat_hbm.at[pl.ds(base + (b + 1) * RB, RB)],
                    bufs[(b + 1) % 2], sems[(b + 1) % 2])
            handles[b % 2].wait()
            buf = bufs[b % 2]

            for s in range(NUM_SEG):
                lo = jnp.clip(bnd[s] - b * RB, 0, RB)
                hi = jnp.clip(bnd[s + 1] - b * RB, 0, RB)
                carry0 = tuple(zeros16 for _ in range(NCH))

                @pl.when(hi > lo)
                def _(s=s, lo=lo, hi=hi, buf=buf, carry0=carry0):
                    def body(r, c):
                        return tuple(
                            c[j] + buf[r, pl.ds(j * 16, 16)]
                            for j in range(NCH))

                    sums_s = plsc.parallel_loop(lo, hi, carry=carry0)(body)
                    for j in range(NCH):
                        acc[s, pl.ds(j * 16, 16)] += sums_s[j]

        pltpu.sync_copy(acc, accs_out.at[wid])

    return k(flat, ids)


def _tc_body(flat_ref, ids_ref, s_ref, c_ref, acc_s, acc_c):
    g = pl.program_id(0)

    @pl.when(g == 0)
    def _():
        acc_s[...] = jnp.zeros((NUM_SEG, D), jnp.float32)
        acc_c[...] = jnp.zeros((NUM_SEG, D), jnp.float32)

    ids = ids_ref[0, 0]
    oh = (lax.broadcasted_iota(jnp.int32, (NUM_SEG, TCB), 0)
          == ids[None, :]).astype(jnp.float32)
    acc_s[...] += jnp.dot(oh, flat_ref[...],
                          preferred_element_type=jnp.float32)
    acc_c[...] += jnp.broadcast_to(
        jnp.sum(oh, axis=1, keepdims=True), (NUM_SEG, D))

    @pl.when(g == NTC - 1)
    def _():
        s_ref[...] = acc_s[...]
        c_ref[...] = acc_c[...]


def _combine_body(part_ref, ts_ref, tcnt_ref, o_ref):
    p = part_ref[...]
    s = ts_ref[...] + jnp.sum(p[:, :, :D], axis=0)
    c = tcnt_ref[...][:, :1] + jnp.sum(p[:, :, D], axis=0)[:, None]
    o_ref[...] = s / jnp.maximum(c, 1.0)


def kernel(flat, segment_ids):
    ids = segment_ids.astype(jnp.int32)
    partials = _sc_partials(flat, ids)
    ids3 = ids.reshape(TOTAL // TCB, 1, TCB)
    tc_sums, tc_cnts = pl.pallas_call(
        _tc_body,
        grid=(NTC,),
        in_specs=[
            pl.BlockSpec((TCB, D), lambda g: (SPLIT // TCB + g, 0)),
            pl.BlockSpec((1, 1, TCB), lambda g: (SPLIT // TCB + g, 0, 0)),
        ],
        out_specs=[
            pl.BlockSpec((NUM_SEG, D), lambda g: (0, 0)),
            pl.BlockSpec((NUM_SEG, D), lambda g: (0, 0)),
        ],
        scratch_shapes=[
            pltpu.VMEM((NUM_SEG, D), jnp.float32),
            pltpu.VMEM((NUM_SEG, D), jnp.float32),
        ],
        out_shape=[
            jax.ShapeDtypeStruct((NUM_SEG, D), jnp.float32),
            jax.ShapeDtypeStruct((NUM_SEG, D), jnp.float32),
        ],
    )(flat, ids3)
    return pl.pallas_call(
        _combine_body,
        out_shape=jax.ShapeDtypeStruct((NUM_SEG, D), jnp.float32),
    )(partials, tc_sums, tc_cnts)


# speedup vs baseline: 1.5816x; 1.1420x over previous
import functools
import math

import jax
import jax.numpy as jnp
from jax import lax
from jax.experimental import pallas as pl
from jax.experimental.pallas import tpu as pltpu
from jax.experimental.pallas import tpu_sc as plsc

NUM_SEG = 16
TOTAL = 32768
D = 128
DW = D + 16
NW = 32

SPLIT = 8192
PER_W = SPLIT // NW
RB = 256
NB = PER_W // RB
NCH = D // 16
NGRP = PER_W // 16
NSTEP = int(math.log2(NGRP))

TCB = 2048
NTC = (TOTAL - SPLIT) // TCB


def _sc_partials(flat, ids):
    mesh = plsc.VectorSubcoreMesh(core_axis_name="c", subcore_axis_name="s")

    @functools.partial(
        pl.kernel,
        out_type=jax.ShapeDtypeStruct((NW, NUM_SEG, DW), jnp.float32),
        mesh=mesh,
        compiler_params=pltpu.CompilerParams(needs_layout_passes=False),
        scratch_types=[
            pltpu.VMEM((PER_W,), jnp.int32),
            pltpu.VMEM((RB, D), jnp.float32),
            pltpu.VMEM((RB, D), jnp.float32),
            pltpu.VMEM((NUM_SEG, DW), jnp.float32),
            pltpu.SemaphoreType.DMA,
            pltpu.SemaphoreType.DMA,
            pltpu.SemaphoreType.DMA,
        ],
    )
    def k(flat_hbm, ids_hbm, accs_out, idv, buf0, buf1, acc,
          sem0, sem1, semi):
        cid = lax.axis_index("c")
        sid = lax.axis_index("s")
        wid = cid * 16 + sid
        base = wid * PER_W

        idcp = pltpu.async_copy(ids_hbm.at[pl.ds(base, PER_W)], idv, semi)

        bufs = (buf0, buf1)
        sems = (sem0, sem1)
        handles = [None, None]
        handles[0] = pltpu.async_copy(
            flat_hbm.at[pl.ds(base, RB)], buf0, sem0)

        zeros16 = jnp.zeros((16,), jnp.float32)
        for s in range(NUM_SEG):
            for j in range(NCH):
                acc[s, pl.ds(j * 16, 16)] = zeros16
        idcp.wait()

        def searchsorted(s):
            lo = jnp.int32(0)
            hi = jnp.int32(NGRP)
            for _ in range(NSTEP):
                mid = (lo + hi) >> 1
                leader = idv[pl.ds(mid * 16, 16)][0]
                pred = leader < s
                lo = jnp.where(pred, mid + 1, lo)
                hi = jnp.where(pred, hi, mid)
            g = jnp.maximum(lo - 1, 0)
            grp = idv[pl.ds(g * 16, 16)]
            cnt = jnp.sum((grp < s).astype(jnp.int32))
            return g * 16 + cnt

        bnd = [jnp.int32(0)]
        for s in range(1, NUM_SEG):
            bnd.append(searchsorted(s))
        bnd.append(jnp.int32(PER_W))

        for s in range(NUM_SEG):
            n = (bnd[s + 1] - bnd[s]).astype(jnp.float32)
            acc[s, pl.ds(D, 16)] = jnp.full((16,), 1.0, jnp.float32) * n

        for b in range(NB):
            if b + 1 < NB:
                handles[(b + 1) % 2] = pltpu.async_copy(
                    flat_hbm.at[pl.ds(base + (b + 1) * RB, RB)],
                    bufs[(b + 1) % 2], sems[(b + 1) % 2])
            handles[b % 2].wait()
            buf = bufs[b % 2]

            for s in range(NUM_SEG):
                lo = jnp.clip(bnd[s] - b * RB, 0, RB)
                hi = jnp.clip(bnd[s + 1] - b * RB, 0, RB)
                carry0 = tuple(zeros16 for _ in range(NCH))

                @pl.when(hi > lo)
                def _(s=s, lo=lo, hi=hi, buf=buf, carry0=carry0):
                    def body(r, c):
                        return tuple(
                            c[j] + buf[r, pl.ds(j * 16, 16)]
                            for j in range(NCH))

                    sums_s = plsc.parallel_loop(lo, hi, carry=carry0)(body)
                    for j in range(NCH):
                        acc[s, pl.ds(j * 16, 16)] += sums_s[j]

        pltpu.sync_copy(acc, accs_out.at[wid])

    return k(flat, ids)


def _tc_body(flat_ref, ids_ref, s_ref, c_ref, acc_s, acc_c):
    g = pl.program_id(0)

    @pl.when(g == 0)
    def _():
        acc_s[...] = jnp.zeros((NUM_SEG, D), jnp.float32)
        acc_c[...] = jnp.zeros((NUM_SEG, D), jnp.float32)

    ids = ids_ref[0, 0]
    oh = (lax.broadcasted_iota(jnp.int32, (NUM_SEG, TCB), 0)
          == ids[None, :]).astype(jnp.float32)
    acc_s[...] += jnp.dot(oh, flat_ref[...],
                          preferred_element_type=jnp.float32)
    acc_c[...] += jnp.broadcast_to(
        jnp.sum(oh, axis=1, keepdims=True), (NUM_SEG, D))

    @pl.when(g == NTC - 1)
    def _():
        s_ref[...] = acc_s[...]
        c_ref[...] = acc_c[...]


def _combine_body(part_ref, ts_ref, tcnt_ref, o_ref):
    p = part_ref[...]
    s = ts_ref[...] + jnp.sum(p[:, :, :D], axis=0)
    c = tcnt_ref[...][:, :1] + jnp.sum(p[:, :, D], axis=0)[:, None]
    o_ref[...] = s / jnp.maximum(c, 1.0)


def kernel(flat, segment_ids):
    ids = segment_ids.astype(jnp.int32)
    partials = _sc_partials(flat, ids)
    ids3 = ids.reshape(TOTAL // TCB, 1, TCB)
    tc_sums, tc_cnts = pl.pallas_call(
        _tc_body,
        grid=(NTC,),
        in_specs=[
            pl.BlockSpec((TCB, D), lambda g: (SPLIT // TCB + g, 0)),
            pl.BlockSpec((1, 1, TCB), lambda g: (SPLIT // TCB + g, 0, 0)),
        ],
        out_specs=[
            pl.BlockSpec((NUM_SEG, D), lambda g: (0, 0)),
            pl.BlockSpec((NUM_SEG, D), lambda g: (0, 0)),
        ],
        scratch_shapes=[
            pltpu.VMEM((NUM_SEG, D), jnp.float32),
            pltpu.VMEM((NUM_SEG, D), jnp.float32),
        ],
        out_shape=[
            jax.ShapeDtypeStruct((NUM_SEG, D), jnp.float32),
            jax.ShapeDtypeStruct((NUM_SEG, D), jnp.float32),
        ],
    )(flat, ids3)
    return pl.pallas_call(
        _combine_body,
        out_shape=jax.ShapeDtypeStruct((NUM_SEG, D), jnp.float32),
    )(partials, tc_sums, tc_cnts)


# speedup vs baseline: 1.6865x; 1.0664x over previous
import functools
import math

import jax
import jax.numpy as jnp
from jax import lax
from jax.experimental import pallas as pl
from jax.experimental.pallas import tpu as pltpu
from jax.experimental.pallas import tpu_sc as plsc

NUM_SEG = 16
TOTAL = 32768
D = 128
DW = D + 16
NW = 32

SPLIT = 4096
PER_W = SPLIT // NW
RB = 128
NB = PER_W // RB
NCH = D // 16
NGRP = PER_W // 16
NSTEP = int(math.log2(NGRP))

TCB = 4096
NTC = (TOTAL - SPLIT) // TCB


def _sc_partials(flat, ids):
    mesh = plsc.VectorSubcoreMesh(core_axis_name="c", subcore_axis_name="s")

    @functools.partial(
        pl.kernel,
        out_type=jax.ShapeDtypeStruct((NW, NUM_SEG, DW), jnp.float32),
        mesh=mesh,
        compiler_params=pltpu.CompilerParams(needs_layout_passes=False),
        scratch_types=[
            pltpu.VMEM((PER_W,), jnp.int32),
            pltpu.VMEM((RB, D), jnp.float32),
            pltpu.VMEM((RB, D), jnp.float32),
            pltpu.VMEM((NUM_SEG, DW), jnp.float32),
            pltpu.SemaphoreType.DMA,
            pltpu.SemaphoreType.DMA,
            pltpu.SemaphoreType.DMA,
        ],
    )
    def k(flat_hbm, ids_hbm, accs_out, idv, buf0, buf1, acc,
          sem0, sem1, semi):
        cid = lax.axis_index("c")
        sid = lax.axis_index("s")
        wid = cid * 16 + sid
        base = wid * PER_W

        idcp = pltpu.async_copy(ids_hbm.at[pl.ds(base, PER_W)], idv, semi)

        bufs = (buf0, buf1)
        sems = (sem0, sem1)
        handles = [None, None]
        handles[0] = pltpu.async_copy(
            flat_hbm.at[pl.ds(base, RB)], buf0, sem0)

        zeros16 = jnp.zeros((16,), jnp.float32)
        for s in range(NUM_SEG):
            for j in range(NCH):
                acc[s, pl.ds(j * 16, 16)] = zeros16
        idcp.wait()

        def searchsorted(s):
            lo = jnp.int32(0)
            hi = jnp.int32(NGRP)
            for _ in range(NSTEP):
                mid = (lo + hi) >> 1
                leader = idv[pl.ds(mid * 16, 16)][0]
                pred = leader < s
                lo = jnp.where(pred, mid + 1, lo)
                hi = jnp.where(pred, hi, mid)
            g = jnp.maximum(lo - 1, 0)
            grp = idv[pl.ds(g * 16, 16)]
            cnt = jnp.sum((grp < s).astype(jnp.int32))
            return g * 16 + cnt

        bnd = [jnp.int32(0)]
        for s in range(1, NUM_SEG):
            bnd.append(searchsorted(s))
        bnd.append(jnp.int32(PER_W))

        for s in range(NUM_SEG):
            n = (bnd[s + 1] - bnd[s]).astype(jnp.float32)
            acc[s, pl.ds(D, 16)] = jnp.full((16,), 1.0, jnp.float32) * n

        for b in range(NB):
            if b + 1 < NB:
                handles[(b + 1) % 2] = pltpu.async_copy(
                    flat_hbm.at[pl.ds(base + (b + 1) * RB, RB)],
                    bufs[(b + 1) % 2], sems[(b + 1) % 2])
            handles[b % 2].wait()
            buf = bufs[b % 2]

            for s in range(NUM_SEG):
                lo = jnp.clip(bnd[s] - b * RB, 0, RB)
                hi = jnp.clip(bnd[s + 1] - b * RB, 0, RB)
                carry0 = tuple(zeros16 for _ in range(NCH))

                @pl.when(hi > lo)
                def _(s=s, lo=lo, hi=hi, buf=buf, carry0=carry0):
                    def body(r, c):
                        return tuple(
                            c[j] + buf[r, pl.ds(j * 16, 16)]
                            for j in range(NCH))

                    sums_s = plsc.parallel_loop(lo, hi, carry=carry0)(body)
                    for j in range(NCH):
                        acc[s, pl.ds(j * 16, 16)] += sums_s[j]

        pltpu.sync_copy(acc, accs_out.at[wid])

    return k(flat, ids)


def _tc_body(flat_ref, ids_ref, s_ref, c_ref, acc_s, acc_c):
    g = pl.program_id(0)

    @pl.when(g == 0)
    def _():
        acc_s[...] = jnp.zeros((NUM_SEG, D), jnp.float32)
        acc_c[...] = jnp.zeros((NUM_SEG, D), jnp.float32)

    ids = ids_ref[0, 0]
    oh = (lax.broadcasted_iota(jnp.int32, (NUM_SEG, TCB), 0)
          == ids[None, :]).astype(jnp.float32)
    acc_s[...] += jnp.dot(oh, flat_ref[...],
                          preferred_element_type=jnp.float32)
    acc_c[...] += jnp.broadcast_to(
        jnp.sum(oh, axis=1, keepdims=True), (NUM_SEG, D))

    @pl.when(g == NTC - 1)
    def _():
        s_ref[...] = acc_s[...]
        c_ref[...] = acc_c[...]


def _combine_body(part_ref, ts_ref, tcnt_ref, o_ref):
    p = part_ref[...]
    s = ts_ref[...] + jnp.sum(p[:, :, :D], axis=0)
    c = tcnt_ref[...][:, :1] + jnp.sum(p[:, :, D], axis=0)[:, None]
    o_ref[...] = s / jnp.maximum(c, 1.0)


def kernel(flat, segment_ids):
    ids = segment_ids.astype(jnp.int32)
    partials = _sc_partials(flat, ids)
    ids3 = ids.reshape(TOTAL // TCB, 1, TCB)
    tc_sums, tc_cnts = pl.pallas_call(
        _tc_body,
        grid=(NTC,),
        in_specs=[
            pl.BlockSpec((TCB, D), lambda g: (SPLIT // TCB + g, 0)),
            pl.BlockSpec((1, 1, TCB), lambda g: (SPLIT // TCB + g, 0, 0)),
        ],
        out_specs=[
            pl.BlockSpec((NUM_SEG, D), lambda g: (0, 0)),
            pl.BlockSpec((NUM_SEG, D), lambda g: (0, 0)),
        ],
        scratch_shapes=[
            pltpu.VMEM((NUM_SEG, D), jnp.float32),
            pltpu.VMEM((NUM_SEG, D), jnp.float32),
        ],
        out_shape=[
            jax.ShapeDtypeStruct((NUM_SEG, D), jnp.float32),
            jax.ShapeDtypeStruct((NUM_SEG, D), jnp.float32),
        ],
    )(flat, ids3)
    return pl.pallas_call(
        _combine_body,
        out_shape=jax.ShapeDtypeStruct((NUM_SEG, D), jnp.float32),
    )(partials, tc_sums, tc_cnts)
